# baseline (speedup 1.0000x reference)
import functools

import jax
import jax.numpy as jnp
from jax import lax
from jax.experimental import pallas as pl
from jax.experimental.pallas import tpu as pltpu
from jax.experimental.pallas import tpu_sc as plsc

_CLS = 1_000_000
_FEAT = 64
_PITCH = 128
_B = 16384
_NC = 2
_NS = 16
_L = 16
_NW = _NC * _NS
_BPW = _B // _NW
_CPS = _B // _NS
_TBL = 1 << 20
_CHK = 64
_NCHK = _BPW // _CHK


def _make_sc_kernel():
    mesh = plsc.VectorSubcoreMesh(core_axis_name="c", subcore_axis_name="s")

    @functools.partial(
        pl.kernel,
        mesh=mesh,
        out_type=jax.ShapeDtypeStruct((_NW * _L,), jnp.float32),
        scratch_types=[
            pltpu.VMEM_SHARED((_TBL,), jnp.float32),
            pltpu.VMEM((_CPS,), jnp.float32),
            pltpu.VMEM((8, 128), jnp.int32),
            pltpu.VMEM((128,), jnp.float32),
            pltpu.VMEM((128,), jnp.float32),
            pltpu.VMEM((_BPW,), jnp.float32),
            pltpu.VMEM((_BPW,), jnp.float32),
            pltpu.VMEM((_CHK, _FEAT), jnp.float32),
            pltpu.VMEM((_CHK, _FEAT), jnp.float32),
            pltpu.VMEM((_CHK, _FEAT), jnp.float32),
            pltpu.VMEM((_CHK, _FEAT), jnp.float32),
            pltpu.VMEM((_L,), jnp.float32),
            pltpu.SemaphoreType.DMA,
            pltpu.SemaphoreType.DMA,
        ],
    )
    def center_loss_sc(xs_h, ys_h, ct_h, out_h,
                       table, yf, cidx, ones, zbuf, cnt, wbuf,
                       xsv0, xsv1, rows0, rows1, outb, sem0, sem1):
        xsvs = (xsv0, xsv1)
        rowss = (rows0, rows1)
        sems = (sem0, sem1)
        c = lax.axis_index("c")
        s = lax.axis_index("s")
        wid = s * _NC + c

        z16 = jnp.zeros((_L,), jnp.float32)
        o16 = jnp.ones((_L,), jnp.float32)

        def ofill(i, carry):
            ones[pl.ds(i * _L, _L)] = o16
            zbuf[pl.ds(i * _L, _L)] = z16
            return carry

        lax.fori_loop(0, 128 // _L, ofill, 0)

        pltpu.sync_copy(ys_h.at[pl.ds(s * _CPS, _CPS)], yf)
        for j in range(8):
            def conv(l, carry, j=j):
                v = yf[pl.ds(j * 128 + l * _L, _L)]
                cidx[j, pl.ds(l * _L, _L)] = v.astype(jnp.int32)
                return carry

            lax.fori_loop(0, 128 // _L, conv, 0)

        for j in range(8):
            pltpu.sync_copy(zbuf, table.at[cidx.at[j]])
        plsc.subcore_barrier()
        for j in range(8):
            pltpu.sync_copy(ones, table.at[cidx.at[j]], add=True)
        plsc.subcore_barrier()

        base = s * _CPS + c * _BPW
        r0 = c * 4
        for j in range(4):
            pltpu.sync_copy(table.at[cidx.at[r0 + j]],
                            cnt.at[pl.ds(j * 128, 128)])
        for g in range(_BPW // _L):
            wbuf[pl.ds(g * _L, _L)] = 0.5 / (cnt[pl.ds(g * _L, _L)] + 1.0)

        def fire(k):
            p = k % 2
            def fire_g(g, carry, k=k, p=p):
                idv = cidx[r0 + k // 2, pl.ds((k % 2) * _CHK + g * _L, _L)]
                for l in range(_L):
                    pltpu.async_copy(ct_h.at[idv[l]],
                                     rowss[p].at[g * _L + l], sems[p])
                return carry

            lax.fori_loop(0, _CHK // _L, fire_g, 0)
            pltpu.async_copy(xs_h.at[pl.ds(base + k * _CHK, _CHK)],
                             xsvs[p], sems[p])

        def drain(k):
            p = k % 2
            pltpu.make_async_copy(ct_h.at[pl.ds(0, _CHK)], rowss[p],
                                  sems[p]).wait()
            pltpu.make_async_copy(xs_h.at[pl.ds(0, _CHK)], xsvs[p],
                                  sems[p]).wait()

        lacc = jnp.zeros((_L,), jnp.float32)
        fire(0)
        for k in range(_NCHK):
            drain(k)
            if k + 1 < _NCHK:
                fire(k + 1)

            p = k % 2

            def group(g, aa, k=k, p=p):
                wv = wbuf[pl.ds(k * _CHK + g * _L, _L)]
                for l in range(_L):
                    w = jnp.full((_L,), wv[l], jnp.float32)
                    e = g * _L + l
                    for v in range(_FEAT // _L):
                        d = (xsvs[p][e, pl.ds(v * _L, _L)]
                             - rowss[p][e, pl.ds(v * _L, _L)])
                        aa = aa + w * (d * d)
                return aa

            lacc = lax.fori_loop(0, _CHK // _L, group, lacc)

        outb[...] = lacc
        pltpu.sync_copy(outb, out_h.at[pl.ds(wid * _L, _L)])

    return center_loss_sc


_center_loss = _make_sc_kernel()


def kernel(xs, ys, center):
    partials = _center_loss(xs, ys, center)
    return jnp.sum(partials)
